# baseline (speedup 1.0000x reference)
import functools

import jax
import jax.numpy as jnp
import numpy as np
from jax import lax
from jax.experimental import pallas as pl
from jax.experimental.pallas import tpu as pltpu
from jax.experimental.pallas import tpu_sc as plsc

_LEVELS = [16, 22, 30, 42, 58, 80, 111, 154, 212, 294, 406, 561, 776, 1073,
           1483, 2048]
_T = 524288
_NLVL = 16
_PI2 = np.int32(np.uint32(2654435761).view(np.int32))
_PI3 = np.int32(805459861)
_HMASK = np.int32(_T - 1)
_NW = 32
_CHUNK = 16
_HALF = 2048


def _lanesel(v, sel):
    return lax.gather(
        v, sel[:, None],
        lax.GatherDimensionNumbers(offset_dims=(), collapsed_slice_dims=(0,),
                                   start_index_map=(0,)),
        (1,), mode=lax.GatherScatterMode.PROMISE_IN_BOUNDS)


def _dup8(v, part):
    return _lanesel(v, (lax.iota(jnp.int32, 16) >> 1) + (8 * part))


def _sc_features(xt, tflat):
    B = xt.shape[1]
    n_per = B // _NW
    n_half = _HALF
    n_sc = n_half // 16
    n_halves = n_per // n_half
    nidx = 8 * _NLVL * 16
    mesh = plsc.VectorSubcoreMesh(core_axis_name="c", subcore_axis_name="s")

    @functools.partial(
        pl.kernel,
        mesh=mesh,
        out_type=jax.ShapeDtypeStruct((2 * _NLVL, B), jnp.float32),
        scratch_types=[
            pltpu.VMEM((3, n_per), jnp.float32),
            pltpu.VMEM((nidx,), jnp.int32),
            pltpu.VMEM((nidx,), jnp.int32),
            pltpu.VMEM((nidx,), jnp.float32),
            pltpu.VMEM((nidx,), jnp.float32),
            pltpu.VMEM((nidx,), jnp.float32),
            pltpu.VMEM((nidx,), jnp.float32),
            pltpu.VMEM((_NLVL * 16,), jnp.float32),
            pltpu.VMEM((2 * _NLVL, n_half), jnp.float32),
            pltpu.SemaphoreType.DMA,
            pltpu.SemaphoreType.DMA,
        ],
    )
    def body(xt_h, tf_h, feat_h, xv, idxa, idxb, wa, wb, va, vb, ta, fv,
             sema, semb):
        wid = lax.axis_index("s") * 2 + lax.axis_index("c")
        base = wid * n_per
        parity = lax.iota(jnp.int32, 16) & 1
        ev_sel = (lax.iota(jnp.int32, 16) << 1) & 15
        od_sel = ev_sel + 1
        lo_half = lax.iota(jnp.int32, 16) < 8
        pltpu.sync_copy(xt_h.at[:, pl.ds(base, n_per)], xv)

        for half in range(n_halves):
            def superchunk(si, carry, half=half):
                s0 = (half * n_sc + si) * 16
                xs0 = xv[0, pl.ds(s0, 16)] / 3.0 + 0.5
                xs1 = xv[1, pl.ds(s0, 16)] / 3.0 + 0.5
                xs2 = xv[2, pl.ds(s0, 16)] / 3.0 + 0.5
                for lvl in range(_NLVL):
                    n = float(_LEVELS[lvl])
                    off = np.int32(lvl * _T)
                    prods = []
                    facs = []
                    for xs, mult in ((xs0, None), (xs1, _PI2), (xs2, _PI3)):
                        xn = xs * n
                        fi = xn.astype(jnp.int32)
                        ff = fi.astype(jnp.float32)
                        fl = jnp.where(xn < ff, fi - 1, fi)
                        fr = xn - fl.astype(jnp.float32)
                        if mult is None:
                            prods.append((fl, fl + 1))
                        else:
                            prods.append((fl * mult, (fl + 1) * mult))
                        facs.append((1.0 - fr, fr))
                    for part, (idxv, wv) in enumerate(((idxa, wa),
                                                       (idxb, wb))):
                        a = [_dup8(prods[0][0], part), _dup8(prods[0][1],
                                                             part)]
                        b = [_dup8(prods[1][0], part), _dup8(prods[1][1],
                                                             part)]
                        c = [_dup8(prods[2][0], part), _dup8(prods[2][1],
                                                             part)]
                        p0 = [_dup8(facs[0][0], part), _dup8(facs[0][1],
                                                             part)]
                        p1 = [_dup8(facs[1][0], part), _dup8(facs[1][1],
                                                             part)]
                        p2 = [_dup8(facs[2][0], part), _dup8(facs[2][1],
                                                             part)]
                        for corner in range(8):
                            wx = corner & 1
                            hy = (corner >> 1) & 1
                            dz = (corner >> 2) & 1
                            h = (a[wx] ^ b[hy] ^ c[dz]) & _HMASK
                            g = ((h + off) << 1) + parity
                            r = (lvl * 8 + corner) * 16
                            idxv[pl.ds(r, 16)] = g
                            wv[pl.ds(r, 16)] = p0[wx] * p1[hy] * p2[dz]
                cpa = pltpu.async_copy(tf_h.at[idxa], va, sema)
                cpb = pltpu.async_copy(tf_h.at[idxb], vb, semb)
                cpa.wait()
                for lvl in range(_NLVL):
                    r0 = lvl * 8 * 16
                    acc = wa[pl.ds(r0, 16)] * va[pl.ds(r0, 16)]
                    for corner in range(1, 8):
                        r = r0 + corner * 16
                        acc = acc + wa[pl.ds(r, 16)] * va[pl.ds(r, 16)]
                    ta[pl.ds(lvl * 16, 16)] = acc
                cpb.wait()
                col = si * 16
                for lvl in range(_NLVL):
                    r0 = lvl * 8 * 16
                    acc = wb[pl.ds(r0, 16)] * vb[pl.ds(r0, 16)]
                    for corner in range(1, 8):
                        r = r0 + corner * 16
                        acc = acc + wb[pl.ds(r, 16)] * vb[pl.ds(r, 16)]
                    acc_a = ta[pl.ds(lvl * 16, 16)]
                    f0 = jnp.where(lo_half, _lanesel(acc_a, ev_sel),
                                   _lanesel(acc, ev_sel))
                    f1 = jnp.where(lo_half, _lanesel(acc_a, od_sel),
                                   _lanesel(acc, od_sel))
                    fv[2 * lvl, pl.ds(col, 16)] = f0
                    fv[2 * lvl + 1, pl.ds(col, 16)] = f1
                return carry

            lax.fori_loop(0, n_sc, superchunk, 0)
            pltpu.sync_copy(
                fv, feat_h.at[:, pl.ds(base + half * n_half, n_half)])

    return body(xt, tflat)


def _tc_mlp(feat, xt, dt, dw1t, db1, dw2t, db2, cw1t, cb1, cw2t, cb2, cw3t,
            cb3):
    B = feat.shape[1]
    bt = 2048
    grid = (B // bt,)

    def body(feat_r, x_r, d_r, dw1_r, db1_r, dw2_r, db2_r, cw1_r, cb1_r,
             cw2_r, cb2_r, cw3_r, cb3_r, out_r):
        f = feat_r[...]
        h1 = jnp.maximum(
            jnp.dot(dw1_r[...], f, preferred_element_type=jnp.float32)
            + db1_r[...], 0.0)
        hd = jnp.dot(dw2_r[...], h1,
                     preferred_element_type=jnp.float32) + db2_r[...]
        xs = x_r[...] / 3.0
        mask = jnp.max(jnp.abs(xs), axis=0, keepdims=True) < 0.5
        log_sigma = jnp.where(mask, hd[0:1, :], -100000.0)
        sigma = jnp.exp(log_sigma)
        db = d_r[...]
        enc = [db]
        for j in range(4):
            s = float(2.0 ** j)
            enc.append(jnp.sin(s * db))
            enc.append(jnp.cos(s * db))
        cin = jnp.concatenate([hd] + enc, axis=0)
        cc = jnp.maximum(
            jnp.dot(cw1_r[...], cin, preferred_element_type=jnp.float32)
            + cb1_r[...], 0.0)
        cc = jnp.maximum(
            jnp.dot(cw2_r[...], cc, preferred_element_type=jnp.float32)
            + cb2_r[...], 0.0)
        z = jnp.dot(cw3_r[...], cc, preferred_element_type=jnp.float32) \
            + cb3_r[...]
        col = 1.0 / (1.0 + jnp.exp(-z))
        col = jnp.where(mask, col, 0.0)
        out_r[...] = jnp.concatenate(
            [col, sigma, jnp.zeros((4, col.shape[1]), jnp.float32)], axis=0)

    wspec = lambda shape: pl.BlockSpec(shape, lambda i: (0, 0))
    return pl.pallas_call(
        body,
        grid=grid,
        in_specs=[
            pl.BlockSpec((32, bt), lambda i: (0, i)),
            pl.BlockSpec((3, bt), lambda i: (0, i)),
            pl.BlockSpec((3, bt), lambda i: (0, i)),
            wspec(dw1t.shape), wspec(db1.shape),
            wspec(dw2t.shape), wspec(db2.shape),
            wspec(cw1t.shape), wspec(cb1.shape),
            wspec(cw2t.shape), wspec(cb2.shape),
            wspec(cw3t.shape), wspec(cb3.shape),
        ],
        out_specs=pl.BlockSpec((8, bt), lambda i: (0, i)),
        out_shape=jax.ShapeDtypeStruct((8, B), jnp.float32),
    )(feat, xt, dt, dw1t, db1, dw2t, db2, cw1t, cb1, cw2t, cb2, cw3t, cb3)


def kernel(x, d, tables, dw1, db1, dw2, db2, cw1, cb1, cw2, cb2, cw3, cb3):
    xt = x.T
    dt = d.T
    tflat = tables.reshape(-1)
    feat = _sc_features(xt, tflat)
    out8 = _tc_mlp(feat, xt, dt,
                   dw1.T, db1.reshape(-1, 1),
                   dw2.T, db2.reshape(-1, 1),
                   cw1.T, cb1.reshape(-1, 1),
                   cw2.T, cb2.reshape(-1, 1),
                   cw3.T, cb3.reshape(-1, 1))
    return out8[:3].T, out8[3]


# speedup vs baseline: 8.6918x; 8.6918x over previous
import functools

import jax
import jax.numpy as jnp
import numpy as np
from jax import lax
from jax.experimental import pallas as pl
from jax.experimental.pallas import tpu as pltpu
from jax.experimental.pallas import tpu_sc as plsc

_LEVELS = [16, 22, 30, 42, 58, 80, 111, 154, 212, 294, 406, 561, 776, 1073,
           1483, 2048]
_T = 524288
_NLVL = 16
_PI2 = np.int32(np.uint32(2654435761).view(np.int32))
_PI3 = np.int32(805459861)
_HMASK = np.int32(_T - 1)
_NW = 32
_CHUNK = 16
_HALF = 2048


def _sc_features(xt, t0, t1):
    B = xt.shape[1]
    n_per = B // _NW
    n_half = _HALF
    n_chunks = n_half // _CHUNK
    n_halves = n_per // n_half
    nidx = 8 * _NLVL * _CHUNK
    mesh = plsc.VectorSubcoreMesh(core_axis_name="c", subcore_axis_name="s")

    @functools.partial(
        pl.kernel,
        mesh=mesh,
        out_type=jax.ShapeDtypeStruct((2 * _NLVL, B), jnp.float32),
        scratch_types=[
            pltpu.VMEM((3, n_per), jnp.float32),
            pltpu.VMEM((nidx,), jnp.int32),
            pltpu.VMEM((nidx,), jnp.int32),
            pltpu.VMEM((nidx,), jnp.float32),
            pltpu.VMEM((nidx,), jnp.float32),
            pltpu.VMEM((nidx,), jnp.float32),
            pltpu.VMEM((nidx,), jnp.float32),
            pltpu.VMEM((nidx,), jnp.float32),
            pltpu.VMEM((nidx,), jnp.float32),
            pltpu.VMEM((2 * _NLVL, n_half), jnp.float32),
            pltpu.SemaphoreType.DMA,
            pltpu.SemaphoreType.DMA,
            pltpu.SemaphoreType.DMA,
            pltpu.SemaphoreType.DMA,
        ],
    )
    def body(xt_h, t0_h, t1_h, feat_h, xv, idxa, idxb, wa, wb, v0a, v1a,
             v0b, v1b, fv, sa0, sa1, sb0, sb1):
        wid = lax.axis_index("s") * 2 + lax.axis_index("c")
        base = wid * n_per
        pltpu.sync_copy(xt_h.at[:, pl.ds(base, n_per)], xv)

        def compute(ci, idxv, wv, half):
            s0 = half * n_half + ci * _CHUNK
            xs0 = xv[0, pl.ds(s0, _CHUNK)] / 3.0 + 0.5
            xs1 = xv[1, pl.ds(s0, _CHUNK)] / 3.0 + 0.5
            xs2 = xv[2, pl.ds(s0, _CHUNK)] / 3.0 + 0.5
            for lvl in range(_NLVL):
                n = float(_LEVELS[lvl])
                off = np.int32(lvl * _T)
                prods = []
                facs = []
                for xs, mult in ((xs0, None), (xs1, _PI2), (xs2, _PI3)):
                    xn = xs * n
                    fi = xn.astype(jnp.int32)
                    ff = fi.astype(jnp.float32)
                    fl = jnp.where(xn < ff, fi - 1, fi)
                    fr = xn - fl.astype(jnp.float32)
                    if mult is None:
                        prods.append((fl, fl + 1))
                    else:
                        prods.append((fl * mult, (fl + 1) * mult))
                    facs.append((1.0 - fr, fr))
                for corner in range(8):
                    wx = corner & 1
                    hy = (corner >> 1) & 1
                    dz = (corner >> 2) & 1
                    h = ((prods[0][wx] ^ prods[1][hy] ^ prods[2][dz])
                         & _HMASK) + off
                    r = (lvl * 8 + corner) * _CHUNK
                    idxv[pl.ds(r, _CHUNK)] = h
                    wv[pl.ds(r, _CHUNK)] = (facs[0][wx] * facs[1][hy]
                                            * facs[2][dz])

        def fire(idxv, v0, v1, s0m, s1m):
            return (pltpu.async_copy(t0_h.at[idxv], v0, s0m),
                    pltpu.async_copy(t1_h.at[idxv], v1, s1m))

        def acc(ci, cps, wv, v0, v1):
            cps[0].wait()
            cps[1].wait()
            for lvl in range(_NLVL):
                r0 = lvl * 8 * _CHUNK
                a0 = wv[pl.ds(r0, _CHUNK)] * v0[pl.ds(r0, _CHUNK)]
                a1 = wv[pl.ds(r0, _CHUNK)] * v1[pl.ds(r0, _CHUNK)]
                for corner in range(1, 8):
                    r = r0 + corner * _CHUNK
                    a0 = a0 + wv[pl.ds(r, _CHUNK)] * v0[pl.ds(r, _CHUNK)]
                    a1 = a1 + wv[pl.ds(r, _CHUNK)] * v1[pl.ds(r, _CHUNK)]
                fv[2 * lvl, pl.ds(ci * _CHUNK, _CHUNK)] = a0
                fv[2 * lvl + 1, pl.ds(ci * _CHUNK, _CHUNK)] = a1

        for half in range(n_halves):
            compute(0, idxa, wa, half)
            cpa = fire(idxa, v0a, v1a, sa0, sa1)

            def pair(k, carry, half=half):
                c0 = 2 * k
                compute(c0 + 1, idxb, wb, half)
                cpb = fire(idxb, v0b, v1b, sb0, sb1)
                acc(c0, cpa, wa, v0a, v1a)
                compute(c0 + 2, idxa, wa, half)
                cpa2 = fire(idxa, v0a, v1a, sa0, sa1)
                acc(c0 + 1, cpb, wb, v0b, v1b)
                return carry

            lax.fori_loop(0, n_chunks // 2 - 1, pair, 0)
            c0 = n_chunks - 2
            compute(c0 + 1, idxb, wb, half)
            cpb = fire(idxb, v0b, v1b, sb0, sb1)
            acc(c0, cpa, wa, v0a, v1a)
            acc(c0 + 1, cpb, wb, v0b, v1b)
            pltpu.sync_copy(
                fv, feat_h.at[:, pl.ds(base + half * n_half, n_half)])

    return body(xt, t0, t1)


def _tc_mlp(feat, xt, dt, dw1t, db1, dw2t, db2, cw1t, cb1, cw2t, cb2, cw3t,
            cb3):
    B = feat.shape[1]
    bt = 2048
    grid = (B // bt,)

    def body(feat_r, x_r, d_r, dw1_r, db1_r, dw2_r, db2_r, cw1_r, cb1_r,
             cw2_r, cb2_r, cw3_r, cb3_r, out_r):
        f = feat_r[...]
        h1 = jnp.maximum(
            jnp.dot(dw1_r[...], f, preferred_element_type=jnp.float32)
            + db1_r[...], 0.0)
        hd = jnp.dot(dw2_r[...], h1,
                     preferred_element_type=jnp.float32) + db2_r[...]
        xs = x_r[...] / 3.0
        mask = jnp.max(jnp.abs(xs), axis=0, keepdims=True) < 0.5
        log_sigma = jnp.where(mask, hd[0:1, :], -100000.0)
        sigma = jnp.exp(log_sigma)
        db = d_r[...]
        enc = [db]
        for j in range(4):
            s = float(2.0 ** j)
            enc.append(jnp.sin(s * db))
            enc.append(jnp.cos(s * db))
        cin = jnp.concatenate([hd] + enc, axis=0)
        cc = jnp.maximum(
            jnp.dot(cw1_r[...], cin, preferred_element_type=jnp.float32)
            + cb1_r[...], 0.0)
        cc = jnp.maximum(
            jnp.dot(cw2_r[...], cc, preferred_element_type=jnp.float32)
            + cb2_r[...], 0.0)
        z = jnp.dot(cw3_r[...], cc, preferred_element_type=jnp.float32) \
            + cb3_r[...]
        col = 1.0 / (1.0 + jnp.exp(-z))
        col = jnp.where(mask, col, 0.0)
        out_r[...] = jnp.concatenate(
            [col, sigma, jnp.zeros((4, col.shape[1]), jnp.float32)], axis=0)

    wspec = lambda shape: pl.BlockSpec(shape, lambda i: (0, 0))
    return pl.pallas_call(
        body,
        grid=grid,
        in_specs=[
            pl.BlockSpec((32, bt), lambda i: (0, i)),
            pl.BlockSpec((3, bt), lambda i: (0, i)),
            pl.BlockSpec((3, bt), lambda i: (0, i)),
            wspec(dw1t.shape), wspec(db1.shape),
            wspec(dw2t.shape), wspec(db2.shape),
            wspec(cw1t.shape), wspec(cb1.shape),
            wspec(cw2t.shape), wspec(cb2.shape),
            wspec(cw3t.shape), wspec(cb3.shape),
        ],
        out_specs=pl.BlockSpec((8, bt), lambda i: (0, i)),
        out_shape=jax.ShapeDtypeStruct((8, B), jnp.float32),
    )(feat, xt, dt, dw1t, db1, dw2t, db2, cw1t, cb1, cw2t, cb2, cw3t, cb3)


def kernel(x, d, tables, dw1, db1, dw2, db2, cw1, cb1, cw2, cb2, cw3, cb3):
    xt = x.T
    dt = d.T
    t0 = tables[:, :, 0].reshape(-1)
    t1 = tables[:, :, 1].reshape(-1)
    feat = _sc_features(xt, t0, t1)
    out8 = _tc_mlp(feat, xt, dt,
                   dw1.T, db1.reshape(-1, 1),
                   dw2.T, db2.reshape(-1, 1),
                   cw1.T, cb1.reshape(-1, 1),
                   cw2.T, cb2.reshape(-1, 1),
                   cw3.T, cb3.reshape(-1, 1))
    return out8[:3].T, out8[3]
